# baseline (speedup 1.0000x reference)
import functools
import math

import jax
import jax.numpy as jnp
from jax import lax
from jax.experimental import pallas as pl
from jax.experimental.pallas import tpu as pltpu
from jax.experimental.pallas import tpu_sc as plsc

D = 32
L = 16
SCALE = math.sqrt(D)
NC, NS = 2, 16
NW = NC * NS
TE, EI = 4, 8
TB, BI = 4, 128
K = TB * BI
JR = TE * TB * EI
JP = BI + 1
RU = 8
NR = 4
NO = 2


def _emb_body(n, xq_hbm, lut_hbm, out_hbm, *refs):
    idxs = refs[0:NR]
    rows = refs[NR:2 * NR]
    obs = refs[2 * NR:2 * NR + NO]
    sis = refs[2 * NR + NO:3 * NR + NO]
    sgs = refs[3 * NR + NO:4 * NR + NO]
    sos = refs[4 * NR + NO:4 * NR + 2 * NO]

    wid = lax.axis_index("s") * NC + lax.axis_index("c")
    tb0 = wid * TB

    def idx_fetch(cc, rb, op):
        for tbl in range(TB):
            src = xq_hbm.at[cc // EI, tb0 + tbl, cc % EI]
            dst = idxs[rb].at[pl.ds(tbl * BI, BI)]
            if op == "sync":
                pltpu.sync_copy(src, dst)
            elif op == "fire":
                pltpu.async_copy(src, dst, sis[rb])
            else:
                pltpu.make_async_copy(src, dst, sis[rb]).wait()

    lane = lax.iota(jnp.int32, L)
    cvec0 = (lane // EI) * (TB * EI) + lane % EI
    cvec1 = cvec0 + 2 * (TB * EI)

    def wb_copy(cc, ob, fire):
        for te in range(TE):
            src = obs[ob].at[pl.ds(te * TB * EI, TB * EI), pl.ds(0, BI)]
            dst = out_hbm.at[cc, te, pl.ds(wid * TB * EI, TB * EI)]
            if fire:
                pltpu.async_copy(src, dst, sos[ob])
            else:
                pltpu.make_async_copy(src, dst, sos[ob]).wait()

    idx_fetch(0, 0, "sync")
    pltpu.async_copy(lut_hbm.at[idxs[0]], rows[0], sgs[0])
    idx_fetch(1, 1, "sync")
    pltpu.async_copy(lut_hbm.at[idxs[1]], rows[1], sgs[1])
    idx_fetch(2, 2, "fire")

    def stage(c, rb, ob):
        g2 = (rb + 2) % NR

        pltpu.make_async_copy(lut_hbm.at[idxs[rb]], rows[rb], sgs[rb]).wait()

        @pl.when(c + 2 < n)
        def _():
            idx_fetch(c + 2, g2, "wait")
            pltpu.async_copy(lut_hbm.at[idxs[g2]], rows[g2], sgs[g2])

        @pl.when(c >= NO)
        def _():
            wb_copy(c - NO, ob, fire=False)

        @plsc.parallel_loop(0, K, unroll=RU)
        def _(r):
            tbl = r // BI
            bi = r % BI
            jv0 = cvec0 + tbl * EI
            jv1 = cvec1 + tbl * EI
            bv = jnp.zeros((L,), jnp.int32) + bi
            v0 = rows[rb][r, pl.ds(0, L)]
            plsc.store_scatter(obs[ob], [jv0, bv], v0 * SCALE)
            v1 = rows[rb][r, pl.ds(L, L)]
            plsc.store_scatter(obs[ob], [jv1, bv], v1 * SCALE)

        wb_copy(c, ob, fire=True)

        @pl.when(c + 3 < n)
        def _():
            idx_fetch(c + 3, (rb + 3) % NR, "fire")

    def outer(o, carry):
        c = o * NR
        for k in range(NR):
            stage(c + k, k, k % NO)
        return carry

    lax.fori_loop(0, n // NR, outer, 0)

    wb_copy(n - 2, 0, fire=False)
    wb_copy(n - 1, 1, fire=False)


def kernel(x, lut):
    B, C = x.shape
    assert B == NW * K and C % NR == 0 and lut.shape[1] == D
    xq = x.T.reshape(C // EI, EI, B // BI, BI).transpose(0, 2, 1, 3)

    mesh = plsc.VectorSubcoreMesh(core_axis_name="c", subcore_axis_name="s")
    f = pl.kernel(
        functools.partial(_emb_body, C),
        out_type=jax.ShapeDtypeStruct((C, TE, (B // BI) * EI, BI), jnp.float32),
        mesh=mesh,
        scratch_types=(
            [pltpu.VMEM((K,), jnp.int32)] * NR
            + [pltpu.VMEM((K, D), jnp.float32)] * NR
            + [pltpu.VMEM((JR, JP), jnp.float32)] * NO
            + [pltpu.SemaphoreType.DMA] * (2 * NR + NO)
        ),
        compiler_params=pltpu.CompilerParams(
            use_tc_tiling_on_sc=False, needs_layout_passes=False),
    )
    o4 = f(xq, lut)
    return (o4.reshape(C, TE, B // BI, EI, BI)
              .transpose((2, 4, 0, 1, 3)).reshape(B, C, D))


# speedup vs baseline: 1.0082x; 1.0082x over previous
import functools
import math

import jax
import jax.numpy as jnp
from jax import lax
from jax.experimental import pallas as pl
from jax.experimental.pallas import tpu as pltpu
from jax.experimental.pallas import tpu_sc as plsc

D = 32
L = 16
SCALE = math.sqrt(D)
NC, NS = 2, 16
NW = NC * NS
TE, EI = 4, 8
TB, BI = 4, 128
K = TB * BI
JR = TE * TB * EI
JP = BI + 1
RU = 4
NR = 4
NO = 2


def _emb_body(n, xq_hbm, lut_hbm, out_hbm, *refs):
    idxs = refs[0:NR]
    rows = refs[NR:2 * NR]
    obs = refs[2 * NR:2 * NR + NO]
    sis = refs[2 * NR + NO:3 * NR + NO]
    sgs = refs[3 * NR + NO:4 * NR + NO]
    sos = refs[4 * NR + NO:4 * NR + 2 * NO]

    wid = lax.axis_index("s") * NC + lax.axis_index("c")
    tb0 = wid * TB

    def idx_fetch(cc, rb, op):
        for tbl in range(TB):
            src = xq_hbm.at[cc // EI, tb0 + tbl, cc % EI]
            dst = idxs[rb].at[pl.ds(tbl * BI, BI)]
            if op == "sync":
                pltpu.sync_copy(src, dst)
            elif op == "fire":
                pltpu.async_copy(src, dst, sis[rb])
            else:
                pltpu.make_async_copy(src, dst, sis[rb]).wait()

    lane = lax.iota(jnp.int32, L)
    cvec0 = (lane // EI) * (TB * EI) + lane % EI
    cvec1 = cvec0 + 2 * (TB * EI)

    def wb_copy(cc, ob, fire):
        for te in range(TE):
            src = obs[ob].at[pl.ds(te * TB * EI, TB * EI), pl.ds(0, BI)]
            dst = out_hbm.at[cc, te, pl.ds(wid * TB * EI, TB * EI)]
            if fire:
                pltpu.async_copy(src, dst, sos[ob])
            else:
                pltpu.make_async_copy(src, dst, sos[ob]).wait()

    idx_fetch(0, 0, "sync")
    pltpu.async_copy(lut_hbm.at[idxs[0]], rows[0], sgs[0])
    idx_fetch(1, 1, "sync")
    pltpu.async_copy(lut_hbm.at[idxs[1]], rows[1], sgs[1])
    idx_fetch(2, 2, "fire")

    def stage(c, rb, ob):
        g2 = (rb + 2) % NR

        pltpu.make_async_copy(lut_hbm.at[idxs[rb]], rows[rb], sgs[rb]).wait()

        @pl.when(c + 2 < n)
        def _():
            idx_fetch(c + 2, g2, "wait")
            pltpu.async_copy(lut_hbm.at[idxs[g2]], rows[g2], sgs[g2])

        @pl.when(c >= NO)
        def _():
            wb_copy(c - NO, ob, fire=False)

        @plsc.parallel_loop(0, K, unroll=RU)
        def _(r):
            tbl = r // BI
            bi = r % BI
            jv0 = cvec0 + tbl * EI
            jv1 = cvec1 + tbl * EI
            bv = jnp.zeros((L,), jnp.int32) + bi
            v0 = rows[rb][r, pl.ds(0, L)]
            plsc.store_scatter(obs[ob], [jv0, bv], v0 * SCALE)
            v1 = rows[rb][r, pl.ds(L, L)]
            plsc.store_scatter(obs[ob], [jv1, bv], v1 * SCALE)

        wb_copy(c, ob, fire=True)

        @pl.when(c + 3 < n)
        def _():
            idx_fetch(c + 3, (rb + 3) % NR, "fire")

    def outer(o, carry):
        c = o * NR
        for k in range(NR):
            stage(c + k, k, k % NO)
        return carry

    lax.fori_loop(0, n // NR, outer, 0)

    wb_copy(n - 2, 0, fire=False)
    wb_copy(n - 1, 1, fire=False)


def kernel(x, lut):
    B, C = x.shape
    assert B == NW * K and C % NR == 0 and lut.shape[1] == D
    xq = x.T.reshape(C // EI, EI, B // BI, BI).transpose(0, 2, 1, 3)

    mesh = plsc.VectorSubcoreMesh(core_axis_name="c", subcore_axis_name="s")
    f = pl.kernel(
        functools.partial(_emb_body, C),
        out_type=jax.ShapeDtypeStruct((C, TE, (B // BI) * EI, BI), jnp.float32),
        mesh=mesh,
        scratch_types=(
            [pltpu.VMEM((K,), jnp.int32)] * NR
            + [pltpu.VMEM((K, D), jnp.float32)] * NR
            + [pltpu.VMEM((JR, JP), jnp.float32)] * NO
            + [pltpu.SemaphoreType.DMA] * (2 * NR + NO)
        ),
        compiler_params=pltpu.CompilerParams(
            use_tc_tiling_on_sc=False, needs_layout_passes=False),
    )
    o4 = f(xq, lut)
    return (o4.reshape(C, TE, B // BI, EI, BI)
              .transpose((2, 4, 0, 1, 3)).reshape(B, C, D))
